# baseline (speedup 1.0000x reference)
import functools

import jax
import jax.numpy as jnp
from jax import lax
from jax.experimental import pallas as pl
from jax.experimental.pallas import tpu as pltpu
from jax.experimental.pallas import tpu_sc as plsc

NN = 10000
EE = 320000
H = 64
GG = 16

NC = 2
NS = 16
NW = NC * NS
NP = 10240
CH = 125
KC = 80
EPT = KC * CH
RPT = NP // NS
RPN = NN // NS

BN = 2000
BE = 8000

_MESH = plsc.VectorSubcoreMesh(core_axis_name="c", subcore_axis_name="s")


def _wid():
    return lax.axis_index("s") * NC + lax.axis_index("c")


NBUF = 8
SET = NBUF // 2
NG = KC // NBUF


@functools.partial(
    pl.kernel,
    out_type=[jax.ShapeDtypeStruct((NC, NP), jnp.float32)],
    mesh=_MESH,
    compiler_params=pltpu.CompilerParams(use_tc_tiling_on_sc=False),
    scratch_types=[
        pltpu.VMEM((KC, CH), jnp.int32),
        pltpu.VMEM((128,), jnp.float32),
        pltpu.VMEM_SHARED((NP,), jnp.float32),
        pltpu.SemaphoreType.DMA,
    ],
)
def _sc_deg(dst_hbm, z1_hbm, deg_hbm, dstv, onesv, dacc, hsem):
    cid = lax.axis_index("c")
    sid = lax.axis_index("s")
    wid = _wid()
    pltpu.sync_copy(dst_hbm.at[wid], dstv)
    for i in range(8):
        onesv[pl.ds(i * 16, 16)] = jnp.ones((16,), jnp.float32)
    pltpu.sync_copy(z1_hbm.at[pl.ds(sid * RPT, RPT)],
                    dacc.at[pl.ds(sid * RPT, RPT)])
    plsc.subcore_barrier()

    def body(g, carry):
        j0 = g * NBUF
        for b in range(NBUF):
            pltpu.async_copy(onesv.at[pl.ds(0, CH)],
                             dacc.at[dstv.at[j0 + b]], hsem, add=True)
        for b in range(NBUF):
            pltpu.make_async_copy(onesv.at[pl.ds(0, CH)],
                                  dacc.at[dstv.at[j0 + b]], hsem).wait()
        return carry

    lax.fori_loop(0, KC // NBUF, body, 0)
    plsc.subcore_barrier()
    pltpu.sync_copy(dacc.at[pl.ds(sid * RPT, RPT)],
                    deg_hbm.at[cid, pl.ds(sid * RPT, RPT)])


@functools.partial(
    pl.kernel,
    out_type=[jax.ShapeDtypeStruct((NC, NN, H), jnp.float32)],
    mesh=_MESH,
    compiler_params=pltpu.CompilerParams(use_tc_tiling_on_sc=False),
    scratch_types=[
        pltpu.VMEM((KC, CH), jnp.int32),
        pltpu.VMEM((KC, CH), jnp.int32),
        pltpu.VMEM((NBUF, CH, H), jnp.float32),
        pltpu.VMEM_SHARED((NN, H), jnp.float32),
    ] + [pltpu.SemaphoreType.DMA] * (2 * NBUF),
)
def _sc_scatter(y_hbm, src_hbm, dst_hbm, z2_hbm, out_hbm,
                srcv, dstv, rows, acc, *sems):
    gsem = sems[:NBUF]
    ssem = sems[NBUF:]
    cid = lax.axis_index("c")
    sid = lax.axis_index("s")
    wid = _wid()
    pltpu.sync_copy(src_hbm.at[wid], srcv)
    pltpu.sync_copy(dst_hbm.at[wid], dstv)
    pltpu.sync_copy(z2_hbm.at[pl.ds(sid * RPN, RPN)],
                    acc.at[pl.ds(sid * RPN, RPN)])
    plsc.subcore_barrier()

    def wait_scatter(j, b):
        pltpu.make_async_copy(rows.at[b], acc.at[dstv.at[j]], ssem[b]).wait()

    def body(g, carry):
        j0 = g * NBUF
        for half in range(2):
            descs = []
            for k in range(SET):
                b = half * SET + k
                j = j0 + b

                @pl.when(g > 0)
                def _(b=b, j=j):
                    wait_scatter(j, b)

                descs.append(
                    pltpu.async_copy(y_hbm.at[srcv.at[j]], rows.at[b],
                                     gsem[b]))
            for k in range(SET):
                b = half * SET + k
                descs[k].wait()
                pltpu.async_copy(rows.at[b], acc.at[dstv.at[j0 + b]],
                                 ssem[b], add=True)
        return carry

    lax.fori_loop(0, NG, body, 0)
    for b in range(NBUF):
        wait_scatter(KC - NBUF + b, b)
    plsc.subcore_barrier()
    pltpu.sync_copy(acc.at[pl.ds(sid * RPN, RPN)],
                    out_hbm.at[cid, pl.ds(sid * RPN, RPN)])


NBG = 4
SETG = NBG // 2


@functools.partial(
    pl.kernel,
    out_type=[
        jax.ShapeDtypeStruct((EE, 128), jnp.float32),
        jax.ShapeDtypeStruct((EE, H), jnp.float32),
    ],
    mesh=_MESH,
    compiler_params=pltpu.CompilerParams(use_tc_tiling_on_sc=False),
    scratch_types=[
        pltpu.VMEM((KC, CH), jnp.int32),
        pltpu.VMEM((KC, CH), jnp.int32),
        pltpu.VMEM((NBG, CH, 128), jnp.float32),
        pltpu.VMEM((NBG, CH, H), jnp.float32),
    ] + [pltpu.SemaphoreType.DMA] * (2 * NBG),
)
def _sc_gather_pairs(a_hbm, b_hbm, src_hbm, dst_hbm, oa_hbm, ob_hbm,
                     srcv, dstv, rows_a, rows_b, *sems):
    gsem = sems[:NBG]
    wsem = sems[NBG:]
    wid = _wid()
    pltpu.sync_copy(src_hbm.at[wid], srcv)
    pltpu.sync_copy(dst_hbm.at[wid], dstv)

    def pipe(tab_hbm, idxv, out_hbm, rows):
        def wait_write(j, b):
            pltpu.make_async_copy(
                rows.at[b], out_hbm.at[pl.ds((wid * KC + j) * CH, CH)],
                wsem[b]).wait()

        def body(g, carry):
            j0 = g * NBG
            for half in range(2):
                descs = []
                for k in range(SETG):
                    b = half * SETG + k
                    j = j0 + b

                    @pl.when(g > 0)
                    def _(b=b, j=j):
                        wait_write(j, b)

                    descs.append(
                        pltpu.async_copy(tab_hbm.at[idxv.at[j]], rows.at[b],
                                         gsem[b]))
                for k in range(SETG):
                    b = half * SETG + k
                    descs[k].wait()
                    pltpu.async_copy(
                        rows.at[b],
                        out_hbm.at[pl.ds((wid * KC + j0 + b) * CH, CH)],
                        wsem[b])
            return carry

        lax.fori_loop(0, KC // NBG, body, 0)
        for b in range(NBG):
            wait_write(KC - NBG + b, b)

    pipe(a_hbm, srcv, oa_hbm, rows_a)
    pipe(b_hbm, dstv, ob_hbm, rows_b)


def _full(shape):
    return pl.BlockSpec(shape, lambda i: tuple(0 for _ in shape))


def _tc_node_body(x_ref, deg_ref, w1, b1, w2, b2, g, be, w0,
                  y0_ref, xw0_ref, dis_ref):
    h = jnp.maximum(x_ref[...] @ w1[...] + b1[...], 0.0)
    h = h @ w2[...] + b2[...]
    m = jnp.mean(h, axis=-1, keepdims=True)
    v = jnp.mean((h - m) ** 2, axis=-1, keepdims=True)
    h = (h - m) * lax.rsqrt(v + 1e-5) * g[...] + be[...]
    deg = deg_ref[...][:, 0:1] + deg_ref[...][:, 1:2] + 1.0
    dis = lax.rsqrt(deg)
    xw = h @ w0[...]
    xw0_ref[...] = xw
    y0_ref[...] = dis * xw
    dis_ref[...] = dis


def _tc_layer1_body(sp_ref, dis_ref, xw0_ref, b0, w1g, y1_ref, xw1_ref):
    s = sp_ref[0] + sp_ref[1]
    dis = dis_ref[...]
    h1 = jnp.maximum(dis * s + dis * dis * xw0_ref[...] + b0[...], 0.0)
    xw1 = h1 @ w1g[...]
    xw1_ref[...] = xw1
    y1_ref[...] = dis * xw1


def _tc_final_body(sp_ref, dis_ref, xw1_ref, b1g, batch_ref,
                   gpw, gpb, gpg, gpbe, epw1, a_ref, b_ref):
    s = sp_ref[0] + sp_ref[1]
    dis = dis_ref[...]
    h2 = dis * s + dis * dis * xw1_ref[...] + b1g[...]
    oh = (batch_ref[...] == lax.broadcasted_iota(jnp.int32, (NN, GG), 1)
          ).astype(jnp.float32)
    ssum = lax.dot_general(oh, h2, dimension_numbers=(((0,), (0,)), ((), ())))
    cnt = jnp.sum(oh, axis=0)
    gmean = ssum / jnp.maximum(cnt, 1.0)[:, None]
    gf = jnp.maximum(gmean @ gpw[...] + gpb[...], 0.0)
    m = jnp.mean(gf, axis=-1, keepdims=True)
    v = jnp.mean((gf - m) ** 2, axis=-1, keepdims=True)
    gf = (gf - m) * lax.rsqrt(v + 1e-5) * gpg[...] + gpbe[...]
    w1 = epw1[...]
    cmat = gf @ w1[128:192, :]
    a_ref[...] = h2 @ w1[0:64, :] + oh @ cmat
    b_ref[...] = h2


def _tc_edge_body(a_ref, hd_ref, ea_ref,
                  ew1, eb1, ew2, eb2, eg, ebe,
                  w1b, w1d, epb1, epw2, epb2, epw3, epb3, o_ref):
    ef = jnp.maximum(ea_ref[...] @ ew1[...] + eb1[...], 0.0)
    ef = ef @ ew2[...] + eb2[...]
    m = jnp.mean(ef, axis=-1, keepdims=True)
    v = jnp.mean((ef - m) ** 2, axis=-1, keepdims=True)
    ef = (ef - m) * lax.rsqrt(v + 1e-5) * eg[...] + ebe[...]
    z = jnp.tanh(a_ref[...] + hd_ref[...] @ w1b[...] + ef @ w1d[...]
                 + epb1[...])
    z = jnp.tanh(z @ epw2[...] + epb2[...])
    o_ref[...] = jax.nn.sigmoid(z @ epw3[...] + epb3[...])


def kernel(x, edge_index, edge_attr, batch, params):
    p = params
    f32 = jnp.float32
    src_p = edge_index[0].reshape(NW, KC, CH)
    dst_p = edge_index[1].reshape(NW, KC, CH)
    z1 = jnp.zeros((NP,), f32)
    z2 = jnp.zeros((NN, H), f32)

    r = lambda a: a.reshape(1, -1)

    (degp,) = _sc_deg(dst_p, z1)
    deg2 = degp[:, :NN].T

    grid_n = NN // BN
    y0, xw0, dis = pl.pallas_call(
        _tc_node_body,
        grid=(grid_n,),
        in_specs=[
            pl.BlockSpec((BN, 128), lambda i: (i, 0)),
            pl.BlockSpec((BN, 2), lambda i: (i, 0)),
            _full((128, H)), _full((1, H)), _full((H, H)), _full((1, H)),
            _full((1, H)), _full((1, H)), _full((H, H)),
        ],
        out_specs=[
            pl.BlockSpec((BN, H), lambda i: (i, 0)),
            pl.BlockSpec((BN, H), lambda i: (i, 0)),
            pl.BlockSpec((BN, 1), lambda i: (i, 0)),
        ],
        out_shape=[
            jax.ShapeDtypeStruct((NN, H), f32),
            jax.ShapeDtypeStruct((NN, H), f32),
            jax.ShapeDtypeStruct((NN, 1), f32),
        ],
    )(x, deg2, p['ne_W1'], r(p['ne_b1']), p['ne_W2'], r(p['ne_b2']),
      r(p['ne_g']), r(p['ne_be']), p['g0_W'])

    (s0,) = _sc_scatter(y0, src_p, dst_p, z2)

    y1, xw1 = pl.pallas_call(
        _tc_layer1_body,
        grid=(grid_n,),
        in_specs=[
            pl.BlockSpec((2, BN, H), lambda i: (0, i, 0)),
            pl.BlockSpec((BN, 1), lambda i: (i, 0)),
            pl.BlockSpec((BN, H), lambda i: (i, 0)),
            _full((1, H)), _full((H, H)),
        ],
        out_specs=[
            pl.BlockSpec((BN, H), lambda i: (i, 0)),
            pl.BlockSpec((BN, H), lambda i: (i, 0)),
        ],
        out_shape=[
            jax.ShapeDtypeStruct((NN, H), f32),
            jax.ShapeDtypeStruct((NN, H), f32),
        ],
    )(s0, dis, xw0, r(p['g0_b']), p['g1_W'])

    (s1,) = _sc_scatter(y1, src_p, dst_p, z2)

    amat, bmat = pl.pallas_call(
        _tc_final_body,
        grid=(1,),
        in_specs=[
            pl.BlockSpec((2, NN, H), lambda i: (0, 0, 0)),
            pl.BlockSpec((NN, 1), lambda i: (0, 0)),
            pl.BlockSpec((NN, H), lambda i: (0, 0)),
            _full((1, H)),
            pl.BlockSpec((NN, 1), lambda i: (0, 0)),
            _full((H, H)), _full((1, H)), _full((1, H)), _full((1, H)),
            _full((256, 128)),
        ],
        out_specs=[
            pl.BlockSpec((NN, 128), lambda i: (0, 0)),
            pl.BlockSpec((NN, H), lambda i: (0, 0)),
        ],
        out_shape=[
            jax.ShapeDtypeStruct((NN, 128), f32),
            jax.ShapeDtypeStruct((NN, H), f32),
        ],
    )(s1, dis, xw1, r(p['g1_b']), batch.reshape(NN, 1),
      p['gp_W'], r(p['gp_b']), r(p['gp_g']), r(p['gp_be']), p['ep_W1'])

    ae, be = _sc_gather_pairs(amat, bmat, src_p, dst_p)

    grid_e = EE // BE
    out = pl.pallas_call(
        _tc_edge_body,
        grid=(grid_e,),
        in_specs=[
            pl.BlockSpec((BE, 128), lambda i: (i, 0)),
            pl.BlockSpec((BE, H), lambda i: (i, 0)),
            pl.BlockSpec((BE, 16), lambda i: (i, 0)),
            _full((16, H)), _full((1, H)), _full((H, H)), _full((1, H)),
            _full((1, H)), _full((1, H)),
            _full((H, 128)), _full((H, 128)), _full((1, 128)),
            _full((128, H)), _full((1, H)),
            _full((H, 1)), _full((1, 1)),
        ],
        out_specs=[pl.BlockSpec((BE, 1), lambda i: (i, 0))],
        out_shape=[jax.ShapeDtypeStruct((EE, 1), f32)],
    )(ae, be, edge_attr,
      p['ee_W1'], r(p['ee_b1']), p['ee_W2'], r(p['ee_b2']),
      r(p['ee_g']), r(p['ee_be']),
      p['ep_W1'][64:128], p['ep_W1'][192:256], r(p['ep_b1']),
      p['ep_W2'], r(p['ep_b2']),
      p['ep_W3'], p['ep_b3'].reshape(1, 1))[0]

    return out


# speedup vs baseline: 1.0945x; 1.0945x over previous
import functools

import jax
import jax.numpy as jnp
from jax import lax
from jax.experimental import pallas as pl
from jax.experimental.pallas import tpu as pltpu
from jax.experimental.pallas import tpu_sc as plsc

NN = 10000
EE = 320000
H = 64
GG = 16

NC = 2
NS = 16
NW = NC * NS
NP = 10240
CH = 125
KC = 80
EPT = KC * CH
RPT = NP // NS
RPN = NN // NS

BN = 2000
BE = 8000

_MESH = plsc.VectorSubcoreMesh(core_axis_name="c", subcore_axis_name="s")


def _wid():
    return lax.axis_index("s") * NC + lax.axis_index("c")


NBUF = 8
SET = NBUF // 2
NG = KC // NBUF


@functools.partial(
    pl.kernel,
    out_type=[jax.ShapeDtypeStruct((NC, NP), jnp.float32)],
    mesh=_MESH,
    compiler_params=pltpu.CompilerParams(use_tc_tiling_on_sc=False),
    scratch_types=[
        pltpu.VMEM((KC, CH), jnp.int32),
        pltpu.VMEM((128,), jnp.float32),
        pltpu.VMEM_SHARED((NP,), jnp.float32),
        pltpu.SemaphoreType.DMA,
    ],
)
def _sc_deg(dst_hbm, z1_hbm, deg_hbm, dstv, onesv, dacc, hsem):
    cid = lax.axis_index("c")
    sid = lax.axis_index("s")
    wid = _wid()
    pltpu.sync_copy(dst_hbm.at[wid], dstv)
    for i in range(8):
        onesv[pl.ds(i * 16, 16)] = jnp.ones((16,), jnp.float32)
    pltpu.sync_copy(z1_hbm.at[pl.ds(sid * RPT, RPT)],
                    dacc.at[pl.ds(sid * RPT, RPT)])
    plsc.subcore_barrier()

    def body(g, carry):
        j0 = g * NBUF
        for b in range(NBUF):
            pltpu.async_copy(onesv.at[pl.ds(0, CH)],
                             dacc.at[dstv.at[j0 + b]], hsem, add=True)
        for b in range(NBUF):
            pltpu.make_async_copy(onesv.at[pl.ds(0, CH)],
                                  dacc.at[dstv.at[j0 + b]], hsem).wait()
        return carry

    lax.fori_loop(0, KC // NBUF, body, 0)
    plsc.subcore_barrier()
    pltpu.sync_copy(dacc.at[pl.ds(sid * RPT, RPT)],
                    deg_hbm.at[cid, pl.ds(sid * RPT, RPT)])


@functools.partial(
    pl.kernel,
    out_type=[jax.ShapeDtypeStruct((NC, NN, H), jnp.float32)],
    mesh=_MESH,
    compiler_params=pltpu.CompilerParams(use_tc_tiling_on_sc=False),
    scratch_types=[
        pltpu.VMEM((KC, CH), jnp.int32),
        pltpu.VMEM((KC, CH), jnp.int32),
        pltpu.VMEM((NBUF, CH, H), jnp.float32),
        pltpu.VMEM_SHARED((NN, H), jnp.float32),
    ] + [pltpu.SemaphoreType.DMA] * (2 * NBUF),
)
def _sc_scatter(y_hbm, src_hbm, dst_hbm, z2_hbm, out_hbm,
                srcv, dstv, rows, acc, *sems):
    gsem = sems[:NBUF]
    ssem = sems[NBUF:]
    cid = lax.axis_index("c")
    sid = lax.axis_index("s")
    wid = _wid()
    pltpu.sync_copy(src_hbm.at[wid], srcv)
    pltpu.sync_copy(dst_hbm.at[wid], dstv)
    pltpu.sync_copy(z2_hbm.at[pl.ds(sid * RPN, RPN)],
                    acc.at[pl.ds(sid * RPN, RPN)])
    plsc.subcore_barrier()

    def wait_scatter(j, b):
        pltpu.make_async_copy(rows.at[b], acc.at[dstv.at[j]], ssem[b]).wait()

    def body(g, carry):
        j0 = g * NBUF
        for half in range(2):
            descs = []
            for k in range(SET):
                b = half * SET + k
                j = j0 + b

                @pl.when(g > 0)
                def _(b=b, j=j):
                    wait_scatter(j, b)

                descs.append(
                    pltpu.async_copy(y_hbm.at[srcv.at[j]], rows.at[b],
                                     gsem[b]))
            for k in range(SET):
                b = half * SET + k
                descs[k].wait()
                pltpu.async_copy(rows.at[b], acc.at[dstv.at[j0 + b]],
                                 ssem[b], add=True)
        return carry

    lax.fori_loop(0, NG, body, 0)
    for b in range(NBUF):
        wait_scatter(KC - NBUF + b, b)
    plsc.subcore_barrier()
    pltpu.sync_copy(acc.at[pl.ds(sid * RPN, RPN)],
                    out_hbm.at[cid, pl.ds(sid * RPN, RPN)])


NBG = 4
SETG = NBG // 2


@functools.partial(
    pl.kernel,
    out_type=[
        jax.ShapeDtypeStruct((EE, 128), jnp.float32),
        jax.ShapeDtypeStruct((EE, 128), jnp.float32),
    ],
    mesh=_MESH,
    compiler_params=pltpu.CompilerParams(use_tc_tiling_on_sc=False),
    scratch_types=[
        pltpu.VMEM((KC, CH), jnp.int32),
        pltpu.VMEM((KC, CH), jnp.int32),
        pltpu.VMEM((NBG, CH, 128), jnp.float32),
    ] + [pltpu.SemaphoreType.DMA] * (2 * NBG),
)
def _sc_gather_pairs(a_hbm, b_hbm, src_hbm, dst_hbm, oa_hbm, ob_hbm,
                     srcv, dstv, rows, *sems):
    gsem = sems[:NBG]
    wsem = sems[NBG:]
    wid = _wid()
    pltpu.sync_copy(src_hbm.at[wid], srcv)
    pltpu.sync_copy(dst_hbm.at[wid], dstv)

    def pipe(tab_hbm, idxv, out_hbm):
        def wait_write(j, b):
            pltpu.make_async_copy(
                rows.at[b], out_hbm.at[pl.ds((wid * KC + j) * CH, CH)],
                wsem[b]).wait()

        def body(g, carry):
            j0 = g * NBG
            for half in range(2):
                descs = []
                for k in range(SETG):
                    b = half * SETG + k
                    j = j0 + b

                    @pl.when(g > 0)
                    def _(b=b, j=j):
                        wait_write(j, b)

                    descs.append(
                        pltpu.async_copy(tab_hbm.at[idxv.at[j]], rows.at[b],
                                         gsem[b]))
                for k in range(SETG):
                    b = half * SETG + k
                    descs[k].wait()
                    pltpu.async_copy(
                        rows.at[b],
                        out_hbm.at[pl.ds((wid * KC + j0 + b) * CH, CH)],
                        wsem[b])
            return carry

        lax.fori_loop(0, KC // NBG, body, 0)
        for b in range(NBG):
            wait_write(KC - NBG + b, b)

    pipe(a_hbm, srcv, oa_hbm)
    pipe(b_hbm, dstv, ob_hbm)


def _full(shape):
    return pl.BlockSpec(shape, lambda i: tuple(0 for _ in shape))


def _tc_node_body(x_ref, deg_ref, w1, b1, w2, b2, g, be, w0,
                  y0_ref, xw0_ref, dis_ref):
    h = jnp.maximum(x_ref[...] @ w1[...] + b1[...], 0.0)
    h = h @ w2[...] + b2[...]
    m = jnp.mean(h, axis=-1, keepdims=True)
    v = jnp.mean((h - m) ** 2, axis=-1, keepdims=True)
    h = (h - m) * lax.rsqrt(v + 1e-5) * g[...] + be[...]
    deg = deg_ref[...][:, 0:1] + deg_ref[...][:, 1:2] + 1.0
    dis = lax.rsqrt(deg)
    xw = h @ w0[...]
    xw0_ref[...] = xw
    y0_ref[...] = dis * xw
    dis_ref[...] = dis


def _tc_layer1_body(sp_ref, dis_ref, xw0_ref, b0, w1g, y1_ref, xw1_ref):
    s = sp_ref[0] + sp_ref[1]
    dis = dis_ref[...]
    h1 = jnp.maximum(dis * s + dis * dis * xw0_ref[...] + b0[...], 0.0)
    xw1 = h1 @ w1g[...]
    xw1_ref[...] = xw1
    y1_ref[...] = dis * xw1


def _tc_final_body(sp_ref, dis_ref, xw1_ref, b1g, batch_ref,
                   gpw, gpb, gpg, gpbe, epw1, a_ref, b_ref):
    s = sp_ref[0] + sp_ref[1]
    dis = dis_ref[...]
    h2 = dis * s + dis * dis * xw1_ref[...] + b1g[...]
    oh = (batch_ref[...] == lax.broadcasted_iota(jnp.int32, (NN, GG), 1)
          ).astype(jnp.float32)
    ssum = lax.dot_general(oh, h2, dimension_numbers=(((0,), (0,)), ((), ())))
    cnt = jnp.sum(oh, axis=0)
    gmean = ssum / jnp.maximum(cnt, 1.0)[:, None]
    gf = jnp.maximum(gmean @ gpw[...] + gpb[...], 0.0)
    m = jnp.mean(gf, axis=-1, keepdims=True)
    v = jnp.mean((gf - m) ** 2, axis=-1, keepdims=True)
    gf = (gf - m) * lax.rsqrt(v + 1e-5) * gpg[...] + gpbe[...]
    w1 = epw1[...]
    cmat = gf @ w1[128:192, :]
    a_ref[...] = h2 @ w1[0:64, :] + oh @ cmat
    b_ref[...] = h2 @ w1[64:128, :]


def _tc_edge_body(a_ref, b_ref, ea_ref,
                  ew1, eb1, ew2, eb2, eg, ebe,
                  w1d, epb1, epw2, epb2, epw3, epb3, o_ref):
    ef = jnp.maximum(ea_ref[...] @ ew1[...] + eb1[...], 0.0)
    ef = ef @ ew2[...] + eb2[...]
    m = jnp.mean(ef, axis=-1, keepdims=True)
    v = jnp.mean((ef - m) ** 2, axis=-1, keepdims=True)
    ef = (ef - m) * lax.rsqrt(v + 1e-5) * eg[...] + ebe[...]
    z = jnp.tanh(a_ref[...] + b_ref[...] + ef @ w1d[...] + epb1[...])
    z = jnp.tanh(z @ epw2[...] + epb2[...])
    o_ref[...] = jax.nn.sigmoid(z @ epw3[...] + epb3[...])


def kernel(x, edge_index, edge_attr, batch, params):
    p = params
    f32 = jnp.float32
    src_p = edge_index[0].reshape(NW, KC, CH)
    dst_p = edge_index[1].reshape(NW, KC, CH)
    z1 = jnp.zeros((NP,), f32)
    z2 = jnp.zeros((NN, H), f32)

    r = lambda a: a.reshape(1, -1)

    (degp,) = _sc_deg(dst_p, z1)
    deg2 = degp[:, :NN].T

    grid_n = NN // BN
    y0, xw0, dis = pl.pallas_call(
        _tc_node_body,
        grid=(grid_n,),
        in_specs=[
            pl.BlockSpec((BN, 128), lambda i: (i, 0)),
            pl.BlockSpec((BN, 2), lambda i: (i, 0)),
            _full((128, H)), _full((1, H)), _full((H, H)), _full((1, H)),
            _full((1, H)), _full((1, H)), _full((H, H)),
        ],
        out_specs=[
            pl.BlockSpec((BN, H), lambda i: (i, 0)),
            pl.BlockSpec((BN, H), lambda i: (i, 0)),
            pl.BlockSpec((BN, 1), lambda i: (i, 0)),
        ],
        out_shape=[
            jax.ShapeDtypeStruct((NN, H), f32),
            jax.ShapeDtypeStruct((NN, H), f32),
            jax.ShapeDtypeStruct((NN, 1), f32),
        ],
    )(x, deg2, p['ne_W1'], r(p['ne_b1']), p['ne_W2'], r(p['ne_b2']),
      r(p['ne_g']), r(p['ne_be']), p['g0_W'])

    (s0,) = _sc_scatter(y0, src_p, dst_p, z2)

    y1, xw1 = pl.pallas_call(
        _tc_layer1_body,
        grid=(grid_n,),
        in_specs=[
            pl.BlockSpec((2, BN, H), lambda i: (0, i, 0)),
            pl.BlockSpec((BN, 1), lambda i: (i, 0)),
            pl.BlockSpec((BN, H), lambda i: (i, 0)),
            _full((1, H)), _full((H, H)),
        ],
        out_specs=[
            pl.BlockSpec((BN, H), lambda i: (i, 0)),
            pl.BlockSpec((BN, H), lambda i: (i, 0)),
        ],
        out_shape=[
            jax.ShapeDtypeStruct((NN, H), f32),
            jax.ShapeDtypeStruct((NN, H), f32),
        ],
    )(s0, dis, xw0, r(p['g0_b']), p['g1_W'])

    (s1,) = _sc_scatter(y1, src_p, dst_p, z2)

    amat, bmat = pl.pallas_call(
        _tc_final_body,
        grid=(1,),
        in_specs=[
            pl.BlockSpec((2, NN, H), lambda i: (0, 0, 0)),
            pl.BlockSpec((NN, 1), lambda i: (0, 0)),
            pl.BlockSpec((NN, H), lambda i: (0, 0)),
            _full((1, H)),
            pl.BlockSpec((NN, 1), lambda i: (0, 0)),
            _full((H, H)), _full((1, H)), _full((1, H)), _full((1, H)),
            _full((256, 128)),
        ],
        out_specs=[
            pl.BlockSpec((NN, 128), lambda i: (0, 0)),
            pl.BlockSpec((NN, 128), lambda i: (0, 0)),
        ],
        out_shape=[
            jax.ShapeDtypeStruct((NN, 128), f32),
            jax.ShapeDtypeStruct((NN, 128), f32),
        ],
    )(s1, dis, xw1, r(p['g1_b']), batch.reshape(NN, 1),
      p['gp_W'], r(p['gp_b']), r(p['gp_g']), r(p['gp_be']), p['ep_W1'])

    ae, be = _sc_gather_pairs(amat, bmat, src_p, dst_p)

    grid_e = EE // BE
    out = pl.pallas_call(
        _tc_edge_body,
        grid=(grid_e,),
        in_specs=[
            pl.BlockSpec((BE, 128), lambda i: (i, 0)),
            pl.BlockSpec((BE, 128), lambda i: (i, 0)),
            pl.BlockSpec((BE, 16), lambda i: (i, 0)),
            _full((16, H)), _full((1, H)), _full((H, H)), _full((1, H)),
            _full((1, H)), _full((1, H)),
            _full((H, 128)), _full((1, 128)),
            _full((128, H)), _full((1, H)),
            _full((H, 1)), _full((1, 1)),
        ],
        out_specs=[pl.BlockSpec((BE, 1), lambda i: (i, 0))],
        out_shape=[jax.ShapeDtypeStruct((EE, 1), f32)],
    )(ae, be, edge_attr,
      p['ee_W1'], r(p['ee_b1']), p['ee_W2'], r(p['ee_b2']),
      r(p['ee_g']), r(p['ee_be']),
      p['ep_W1'][192:256], r(p['ep_b1']), p['ep_W2'], r(p['ep_b2']),
      p['ep_W3'], p['ep_b3'].reshape(1, 1))[0]

    return out
